# Initial kernel scaffold; baseline (speedup 1.0000x reference)
import jax
import jax.numpy as jnp
from jax.experimental import pallas as pl


def kernel(x, edge_index, enc_W0, enc_b0, enc_W1, enc_b1, dec_W0, dec_b0, dec_W1, dec_b1):
    raise NotImplementedError("write your pallas kernel here")


# speedup vs baseline: 9.1756x; 9.1756x over previous
import functools

import jax
import jax.numpy as jnp
from jax import lax
from jax.experimental import pallas as pl
from jax.experimental.pallas import tpu as pltpu
from jax.experimental.pallas import tpu_sc as plsc

N = 10000
N_PAD = 10240
NW = 32
CHUNK = 128
STEPS = 79
E_PAD = NW * STEPS * CHUNK
RPS = N_PAD // 16

_MESH = dict(core_axis_name="c", subcore_axis_name="s")


def _make_deg_kernel():
  @functools.partial(
      pl.kernel,
      out_type=jax.ShapeDtypeStruct((2, N_PAD), jnp.float32),
      mesh=plsc.VectorSubcoreMesh(**_MESH),
      scratch_types=[
          pltpu.VMEM((STEPS, CHUNK), jnp.int32),
          pltpu.VMEM((CHUNK,), jnp.float32),
          pltpu.VMEM_SHARED((N_PAD,), jnp.float32),
      ],
  )
  def deg_kernel(dst_hbm, zeros_hbm, out_hbm, didx_v, ones_v, acc_sh):
    c = lax.axis_index("c")
    s = lax.axis_index("s")
    w = s * 2 + c
    pltpu.sync_copy(dst_hbm.at[w], didx_v)
    for i in range(CHUNK // 16):
      ones_v[pl.ds(i * 16, 16)] = jnp.ones((16,), jnp.float32)
    pltpu.sync_copy(zeros_hbm.at[pl.ds(s * RPS, RPS)],
                    acc_sh.at[pl.ds(s * RPS, RPS)])
    plsc.subcore_barrier()

    def step(j, carry):
      pltpu.sync_copy(ones_v, acc_sh.at[didx_v.at[j]], add=True)
      return carry

    lax.fori_loop(0, STEPS, step, 0)
    plsc.subcore_barrier()
    pltpu.sync_copy(acc_sh.at[pl.ds(s * RPS, RPS)],
                    out_hbm.at[c, pl.ds(s * RPS, RPS)])

  return deg_kernel


def _make_scatter_kernel(width):
  @functools.partial(
      pl.kernel,
      out_type=jax.ShapeDtypeStruct((2, N_PAD, width), jnp.float32),
      mesh=plsc.VectorSubcoreMesh(**_MESH),
      scratch_types=[
          pltpu.VMEM((STEPS, CHUNK), jnp.int32),
          pltpu.VMEM((STEPS, CHUNK), jnp.int32),
          pltpu.VMEM((CHUNK, width), jnp.float32),
          pltpu.VMEM_SHARED((N_PAD, width), jnp.float32),
          pltpu.SemaphoreType.DMA,
      ],
      compiler_params=pltpu.CompilerParams(use_tc_tiling_on_sc=False),
  )
  def scat_kernel(src_hbm, dst_hbm, y_hbm, zeros_hbm, out_hbm,
                  sidx_v, didx_v, rows_v, acc_sh, sem):
    c = lax.axis_index("c")
    s = lax.axis_index("s")
    w = s * 2 + c
    pltpu.sync_copy(src_hbm.at[w], sidx_v)
    pltpu.sync_copy(dst_hbm.at[w], didx_v)
    pltpu.sync_copy(zeros_hbm.at[pl.ds(s * RPS, RPS)],
                    acc_sh.at[pl.ds(s * RPS, RPS)])
    plsc.subcore_barrier()

    def step(j, carry):
      pltpu.async_copy(y_hbm.at[sidx_v.at[j]], rows_v, sem).wait()
      pltpu.sync_copy(rows_v, acc_sh.at[didx_v.at[j]], add=True)
      return carry

    lax.fori_loop(0, STEPS, step, 0)
    plsc.subcore_barrier()
    pltpu.sync_copy(acc_sh.at[pl.ds(s * RPS, RPS)],
                    out_hbm.at[c, pl.ds(s * RPS, RPS)])

  return scat_kernel


_DEG = _make_deg_kernel()
_SCATTER = _make_scatter_kernel(128)


def _dinv_body(da_ref, db_ref, o_ref):
  o_ref[...] = lax.rsqrt(da_ref[...] + db_ref[...] + 1.0)


def _tc_dinv(dega, degb):
  return pl.pallas_call(
      _dinv_body,
      out_shape=jax.ShapeDtypeStruct((N_PAD // 128, 128), jnp.float32),
  )(dega.reshape(N_PAD // 128, 128), degb.reshape(N_PAD // 128, 128))


def _mm_body(dinv_ref, h_ref, w_ref, y_ref):
  y_ref[...] = dinv_ref[...] * jnp.dot(
      h_ref[...], w_ref[...], preferred_element_type=jnp.float32)


def _tc_matmul(h, W, dinv_col):
  n, din = h.shape
  dout = W.shape[1]
  blk = 1024
  return pl.pallas_call(
      _mm_body,
      grid=(n // blk,),
      in_specs=[
          pl.BlockSpec((blk, 1), lambda i: (i, 0)),
          pl.BlockSpec((blk, din), lambda i: (i, 0)),
          pl.BlockSpec((din, dout), lambda i: (0, 0)),
      ],
      out_specs=pl.BlockSpec((blk, dout), lambda i: (i, 0)),
      out_shape=jax.ShapeDtypeStruct((n, dout), jnp.float32),
  )(dinv_col, h, W)


def _comb_body(dinv_ref, aa_ref, ab_ref, y_ref, b_ref, o_ref):
  o_ref[...] = jnp.maximum(
      dinv_ref[...] * (aa_ref[...] + ab_ref[...] + y_ref[...]) + b_ref[...],
      0.0)


def _tc_combine(acca, accb, y, dinv_col, b):
  n, dout = y.shape
  blk = 1024
  return pl.pallas_call(
      _comb_body,
      grid=(n // blk,),
      in_specs=[
          pl.BlockSpec((blk, 1), lambda i: (i, 0)),
          pl.BlockSpec((blk, dout), lambda i: (i, 0)),
          pl.BlockSpec((blk, dout), lambda i: (i, 0)),
          pl.BlockSpec((blk, dout), lambda i: (i, 0)),
          pl.BlockSpec((1, dout), lambda i: (0, 0)),
      ],
      out_specs=pl.BlockSpec((blk, dout), lambda i: (i, 0)),
      out_shape=jax.ShapeDtypeStruct((n, dout), jnp.float32),
  )(dinv_col, acca, accb, y, b)


def kernel(x, edge_index, enc_W0, enc_b0, enc_W1, enc_b1,
           dec_W0, dec_b0, dec_W1, dec_b1):
  src = edge_index[0]
  dst = edge_index[1]
  pad = E_PAD - src.shape[0]
  src3 = jnp.concatenate(
      [src, jnp.zeros((pad,), jnp.int32)]).reshape(NW, STEPS, CHUNK)
  dst3 = jnp.concatenate(
      [dst, jnp.full((pad,), N, jnp.int32)]).reshape(NW, STEPS, CHUNK)
  zeros1 = jnp.zeros((N_PAD,), jnp.float32)
  zeros2 = jnp.zeros((N_PAD, 128), jnp.float32)

  deg = _DEG(dst3, zeros1)
  dinv_col = _tc_dinv(deg[0], deg[1]).reshape(N_PAD, 1)

  h = jnp.pad(x, ((0, N_PAD - N), (0, 0)))
  for W, b in ((enc_W0, enc_b0), (enc_W1, enc_b1),
               (dec_W0, dec_b0), (dec_W1, dec_b1)):
    Wp = jnp.pad(W, ((0, 128 - W.shape[0]), (0, 128 - W.shape[1])))
    bp = jnp.pad(b, (0, 128 - b.shape[0]))
    y = _tc_matmul(h, Wp, dinv_col)
    accs = _SCATTER(src3, dst3, y, zeros2)
    h = _tc_combine(accs[0], accs[1], y, dinv_col, bp.reshape(1, 128))
  return h[:N]
